# baseline (speedup 1.0000x reference)
import functools

import jax
import jax.numpy as jnp
from jax import lax
from jax.experimental import pallas as pl
from jax.experimental.pallas import tpu as pltpu
from jax.experimental.pallas import tpu_sc as plsc

_EMB = 128
_HID = 512
_CHUNK = 104
_NCHUNK = 2
_ROWS_W = _CHUNK * _NCHUNK


def _emb_gather(table, idx3):
    nw = idx3.shape[0]
    mesh = plsc.VectorSubcoreMesh(core_axis_name="c", subcore_axis_name="s")

    @functools.partial(
        pl.kernel,
        mesh=mesh,
        out_type=jax.ShapeDtypeStruct((nw * _ROWS_W, _EMB), jnp.float32),
        scratch_types=[
            pltpu.VMEM((_NCHUNK, _CHUNK), jnp.int32),
            pltpu.VMEM((_ROWS_W, _EMB), jnp.float32),
            pltpu.SemaphoreType.DMA,
        ],
    )
    def gather_kernel(table_hbm, idx_hbm, out_hbm, idx_v, rows_v, sem):
        nc = lax.axis_size("c")
        wid = lax.axis_index("s") * nc + lax.axis_index("c")
        pltpu.sync_copy(idx_hbm.at[wid], idx_v)
        cps = []
        for j in range(_NCHUNK):
            cps.append(
                pltpu.async_copy(
                    table_hbm.at[idx_v.at[j]],
                    rows_v.at[pl.ds(j * _CHUNK, _CHUNK)],
                    sem,
                )
            )
        for cp in cps:
            cp.wait()
        pltpu.sync_copy(rows_v, out_hbm.at[pl.ds(wid * _ROWS_W, _ROWS_W)])

    return gather_kernel(table, idx3)


def _cell(a_bf16, c_prev, w_ref, b_ref):
    g = jnp.dot(a_bf16, w_ref[...], preferred_element_type=jnp.float32)
    g = g + b_ref[...]
    i = jax.nn.sigmoid(g[:, :_HID])
    f = jax.nn.sigmoid(g[:, _HID:2 * _HID])
    u = jnp.tanh(g[:, 2 * _HID:3 * _HID])
    o = jax.nn.sigmoid(g[:, 3 * _HID:])
    c = f * c_prev + i * u
    h = o * jnp.tanh(c)
    return h, c


def _lstm2_body(net_ref, len_ref, eos_ref, w0_ref, w1_ref, b0_ref, b1_ref,
                ys_ref, h0_ref, c0_ref, h1_ref, c1_ref, *, t_last):
    b_sz = h0_ref.shape[0]
    z = jnp.zeros((b_sz, _HID), jnp.float32)

    def pair(k, carry):
        h0m2, h0m1, c0_prev, h1_prev, c1_prev = carry
        u0 = 2 * k
        u1 = 2 * k + 1
        s0 = 2 * k - 2
        s1 = 2 * k - 1

        xa = net_ref[pl.ds(jnp.minimum(u0, t_last), 1)][0]
        xa = jnp.where(len_ref[...] == u0, eos_ref[...], xa)
        hA, cA = _cell(
            jnp.concatenate([xa, h0m1], axis=1).astype(jnp.bfloat16),
            c0_prev, w0_ref, b0_ref)
        xb = net_ref[pl.ds(jnp.minimum(u1, t_last), 1)][0]
        xb = jnp.where(len_ref[...] == u1, eos_ref[...], xb)
        hB, cB = _cell(
            jnp.concatenate([xb, hA], axis=1).astype(jnp.bfloat16),
            cA, w0_ref, b0_ref)

        h1A, c1A = _cell(
            jnp.concatenate([h0m2, h1_prev], axis=1).astype(jnp.bfloat16),
            c1_prev, w1_ref, b1_ref)
        h1B, c1B = _cell(
            jnp.concatenate([h0m1, h1A], axis=1).astype(jnp.bfloat16),
            c1A, w1_ref, b1_ref)
        ys_ref[pl.ds(jnp.maximum(s0, 0), 1)] = h1A[None]
        ys_ref[pl.ds(jnp.maximum(s1, 0), 1)] = h1B[None]

        k0a = u0 <= t_last
        k0b = u1 <= t_last
        k1a = (s0 >= 0) & (s0 <= t_last)
        k1b = (s1 >= 0) & (s1 <= t_last)
        return (jnp.where(k0a, hA, h0m2),
                jnp.where(k0b, hB, jnp.where(k0a, hA, h0m1)),
                jnp.where(k0b, cB, jnp.where(k0a, cA, c0_prev)),
                jnp.where(k1b, h1B, jnp.where(k1a, h1A, h1_prev)),
                jnp.where(k1b, c1B, jnp.where(k1a, c1A, c1_prev)))

    n_pairs = t_last // 2 + 1
    h0m2, _, c0, h1p, c1p = lax.fori_loop(0, n_pairs, pair, (z, z, z, z, z))
    h1, c1 = _cell(
        jnp.concatenate([h0m2, h1p], axis=1).astype(jnp.bfloat16),
        c1p, w1_ref, b1_ref)
    ys_ref[pl.ds(t_last, 1)] = h1[None]
    h0_ref[...] = h0m2
    c0_ref[...] = c0
    h1_ref[...] = h1
    c1_ref[...] = c1


def _lstm2(net, lengths2d, eos_row, w0, w1, b0, b1, *, interpret=False):
    tp1, b_sz, _ = net.shape
    t_last = tp1 - 1
    out_shape = [
        jax.ShapeDtypeStruct((tp1, b_sz, _HID), jnp.float32),
        jax.ShapeDtypeStruct((b_sz, _HID), jnp.float32),
        jax.ShapeDtypeStruct((b_sz, _HID), jnp.float32),
        jax.ShapeDtypeStruct((b_sz, _HID), jnp.float32),
        jax.ShapeDtypeStruct((b_sz, _HID), jnp.float32),
    ]
    return pl.pallas_call(
        functools.partial(_lstm2_body, t_last=t_last),
        out_shape=out_shape,
        interpret=interpret,
    )(net, lengths2d, eos_row, w0, w1, b0, b1)


def kernel(inputs, lengths, emb, Wih0, Whh0, bih0, bhh0, Wih1, Whh1, bih1, bhh1):
    t_sz, b_sz = inputs.shape
    vocab = emb.shape[0] - 1

    x = jnp.concatenate(
        [inputs.astype(jnp.int32), jnp.zeros((1, b_sz), jnp.int32)], axis=0)
    idx_flat = x.reshape(-1)
    n_tok = idx_flat.shape[0]

    info = plsc.get_sparse_core_info()
    nw = info.num_cores * info.num_subcores
    n_pad = nw * _ROWS_W
    idx_pad = jnp.concatenate(
        [idx_flat, jnp.zeros((n_pad - n_tok,), jnp.int32)])
    idx3 = idx_pad.reshape(nw, _NCHUNK, _CHUNK)

    rows = _emb_gather(emb, idx3)
    net = rows[:n_tok].reshape(t_sz + 1, b_sz, _EMB)

    w0 = jnp.concatenate([Wih0, Whh0], axis=1).T.astype(jnp.bfloat16)
    w1 = jnp.concatenate([Wih1, Whh1], axis=1).T.astype(jnp.bfloat16)
    b0 = (bih0 + bhh0).reshape(1, -1)
    b1 = (bih1 + bhh1).reshape(1, -1)
    lengths2d = lengths.astype(jnp.int32).reshape(b_sz, 1)
    eos_row = emb[vocab].reshape(1, _EMB)

    ys1, h0, c0, h1, c1 = _lstm2(net, lengths2d, eos_row, w0, w1, b0, b1)
    hN = jnp.stack([h0, h1], axis=0)
    cN = jnp.stack([c0, c1], axis=0)
    return ys1, hN, cN


# speedup vs baseline: 1.0062x; 1.0062x over previous
import functools

import jax
import jax.numpy as jnp
from jax import lax
from jax.experimental import pallas as pl
from jax.experimental.pallas import tpu as pltpu
from jax.experimental.pallas import tpu_sc as plsc

_EMB = 128
_HID = 512
_CHUNK = 104
_NCHUNK = 2
_ROWS_W = _CHUNK * _NCHUNK


def _emb_gather(table, idx3):
    nw = idx3.shape[0]
    mesh = plsc.VectorSubcoreMesh(core_axis_name="c", subcore_axis_name="s")

    @functools.partial(
        pl.kernel,
        mesh=mesh,
        out_type=jax.ShapeDtypeStruct((nw * _ROWS_W, _EMB), jnp.float32),
        scratch_types=[
            pltpu.VMEM((_NCHUNK, _CHUNK), jnp.int32),
            pltpu.VMEM((_ROWS_W, _EMB), jnp.float32),
            pltpu.SemaphoreType.DMA,
        ],
    )
    def gather_kernel(table_hbm, idx_hbm, out_hbm, idx_v, rows_v, sem):
        nc = lax.axis_size("c")
        wid = lax.axis_index("s") * nc + lax.axis_index("c")
        pltpu.sync_copy(idx_hbm.at[wid], idx_v)
        cps = []
        for j in range(_NCHUNK):
            cps.append(
                pltpu.async_copy(
                    table_hbm.at[idx_v.at[j]],
                    rows_v.at[pl.ds(j * _CHUNK, _CHUNK)],
                    sem,
                )
            )
        for cp in cps:
            cp.wait()
        pltpu.sync_copy(rows_v, out_hbm.at[pl.ds(wid * _ROWS_W, _ROWS_W)])

    return gather_kernel(table, idx3)


def _cell(a_bf16, c_prev, w_ref, b_ref):
    g = jnp.dot(a_bf16, w_ref[...], preferred_element_type=jnp.float32)
    g = g + b_ref[...]
    i = jax.nn.sigmoid(g[:, :_HID])
    f = jax.nn.sigmoid(g[:, _HID:2 * _HID])
    u = jnp.tanh(g[:, 2 * _HID:3 * _HID])
    o = jax.nn.sigmoid(g[:, 3 * _HID:])
    c = f * c_prev + i * u
    h = o * jnp.tanh(c)
    return h, c


def _lstm2_body(net_ref, len_ref, eos_ref, w0_ref, w1_ref, b0_ref, b1_ref,
                ys_ref, hn_ref, cn_ref, *, t_last):
    b_sz = hn_ref.shape[1]
    z = jnp.zeros((b_sz, _HID), jnp.float32)

    def pair(k, carry):
        h0m2, h0m1, c0_prev, h1_prev, c1_prev = carry
        u0 = 2 * k
        u1 = 2 * k + 1
        s0 = 2 * k - 2
        s1 = 2 * k - 1

        xa = net_ref[pl.ds(jnp.minimum(u0, t_last), 1)][0]
        xa = jnp.where(len_ref[...] == u0, eos_ref[...], xa)
        hA, cA = _cell(
            jnp.concatenate([xa, h0m1], axis=1).astype(jnp.bfloat16),
            c0_prev, w0_ref, b0_ref)
        xb = net_ref[pl.ds(jnp.minimum(u1, t_last), 1)][0]
        xb = jnp.where(len_ref[...] == u1, eos_ref[...], xb)
        hB, cB = _cell(
            jnp.concatenate([xb, hA], axis=1).astype(jnp.bfloat16),
            cA, w0_ref, b0_ref)

        h1A, c1A = _cell(
            jnp.concatenate([h0m2, h1_prev], axis=1).astype(jnp.bfloat16),
            c1_prev, w1_ref, b1_ref)
        h1B, c1B = _cell(
            jnp.concatenate([h0m1, h1A], axis=1).astype(jnp.bfloat16),
            c1A, w1_ref, b1_ref)
        ys_ref[pl.ds(jnp.maximum(s0, 0), 1)] = h1A[None]
        ys_ref[pl.ds(jnp.maximum(s1, 0), 1)] = h1B[None]

        k0a = u0 <= t_last
        k0b = u1 <= t_last
        k1a = (s0 >= 0) & (s0 <= t_last)
        k1b = (s1 >= 0) & (s1 <= t_last)
        return (jnp.where(k0a, hA, h0m2),
                jnp.where(k0b, hB, jnp.where(k0a, hA, h0m1)),
                jnp.where(k0b, cB, jnp.where(k0a, cA, c0_prev)),
                jnp.where(k1b, h1B, jnp.where(k1a, h1A, h1_prev)),
                jnp.where(k1b, c1B, jnp.where(k1a, c1A, c1_prev)))

    n_pairs = t_last // 2 + 1
    h0m2, _, c0, h1p, c1p = lax.fori_loop(0, n_pairs, pair, (z, z, z, z, z))
    h1, c1 = _cell(
        jnp.concatenate([h0m2, h1p], axis=1).astype(jnp.bfloat16),
        c1p, w1_ref, b1_ref)
    ys_ref[pl.ds(t_last, 1)] = h1[None]
    hn_ref[0] = h0m2
    hn_ref[1] = h1
    cn_ref[0] = c0
    cn_ref[1] = c1


def _lstm2(net, lengths2d, eos_row, w0, w1, b0, b1, *, interpret=False):
    tp1, b_sz, _ = net.shape
    t_last = tp1 - 1
    out_shape = [
        jax.ShapeDtypeStruct((tp1, b_sz, _HID), jnp.float32),
        jax.ShapeDtypeStruct((2, b_sz, _HID), jnp.float32),
        jax.ShapeDtypeStruct((2, b_sz, _HID), jnp.float32),
    ]
    return pl.pallas_call(
        functools.partial(_lstm2_body, t_last=t_last),
        out_shape=out_shape,
        interpret=interpret,
    )(net, lengths2d, eos_row, w0, w1, b0, b1)


def kernel(inputs, lengths, emb, Wih0, Whh0, bih0, bhh0, Wih1, Whh1, bih1, bhh1):
    t_sz, b_sz = inputs.shape
    vocab = emb.shape[0] - 1

    x = jnp.concatenate(
        [inputs.astype(jnp.int32), jnp.zeros((1, b_sz), jnp.int32)], axis=0)
    idx_flat = x.reshape(-1)
    n_tok = idx_flat.shape[0]

    info = plsc.get_sparse_core_info()
    nw = info.num_cores * info.num_subcores
    n_pad = nw * _ROWS_W
    idx_pad = jnp.concatenate(
        [idx_flat, jnp.zeros((n_pad - n_tok,), jnp.int32)])
    idx3 = idx_pad.reshape(nw, _NCHUNK, _CHUNK)

    rows = _emb_gather(emb, idx3)
    net = rows[:n_tok].reshape(t_sz + 1, b_sz, _EMB)

    w0 = jnp.concatenate([Wih0, Whh0], axis=1).astype(jnp.bfloat16).T
    w1 = jnp.concatenate([Wih1, Whh1], axis=1).astype(jnp.bfloat16).T
    b0 = (bih0 + bhh0).reshape(1, -1)
    b1 = (bih1 + bhh1).reshape(1, -1)
    lengths2d = lengths.astype(jnp.int32).reshape(b_sz, 1)
    eos_row = emb[vocab].reshape(1, _EMB)

    ys1, hN, cN = _lstm2(net, lengths2d, eos_row, w0, w1, b0, b1)
    return ys1, hN, cN
